# Initial kernel scaffold; baseline (speedup 1.0000x reference)
import jax
import jax.numpy as jnp
from jax.experimental import pallas as pl


def kernel(query, key, value, proj_dir):
    raise NotImplementedError("write your pallas kernel here")


# speedup vs baseline: 5.1492x; 5.1492x over previous
import math

import jax
import jax.numpy as jnp
from jax import lax
from jax.experimental import pallas as pl
from jax.experimental.pallas import tpu as pltpu

B, H, N, E = 2, 16, 4096, 64
NUM_PROJS = 7
BUCKET_SIZE = 64
SAMPLE_SIZE = 256
BH = B * H
TQ = 512
G = 128


def _attn_body(qs_ref, ks_ref, vs_ref, kpi_ref, vpi_ref, sig_ref, blk_ref,
               out_ref):
    t = pl.program_id(1)
    q = qs_ref[0]
    k = ks_ref[0]
    v = vs_ref[0]
    kpi = kpi_ref[0]
    vpi = vpi_ref[0]
    sig = sig_ref[0]
    blk = blk_ref[0]

    s2 = jnp.dot(q, kpi.T, preferred_element_type=jnp.float32)
    r_blk = t * (TQ // BUCKET_SIZE) + (
        lax.broadcasted_iota(jnp.int32, (TQ, SAMPLE_SIZE), 0) // BUCKET_SIZE)
    keep = r_blk != blk
    a2 = jnp.where(keep, jnp.exp(s2), 0.0) * sig
    num = jnp.dot(a2, vpi, preferred_element_type=jnp.float32)
    den = jnp.sum(a2, axis=1, keepdims=True)

    nums = []
    dens = []
    ri = lax.broadcasted_iota(jnp.int32, (G, G), 0) // BUCKET_SIZE
    ci = lax.broadcasted_iota(jnp.int32, (G, G), 1) // BUCKET_SIZE
    local = ri == ci
    for g in range(TQ // G):
        qg = q[g * G:(g + 1) * G]
        kg = k[g * G:(g + 1) * G]
        vg = v[g * G:(g + 1) * G]
        s1 = jnp.dot(qg, kg.T, preferred_element_type=jnp.float32)
        a1 = jnp.where(local, jnp.exp(s1), 0.0)
        nums.append(jnp.dot(a1, vg, preferred_element_type=jnp.float32))
        dens.append(jnp.sum(a1, axis=1, keepdims=True))
    num = num + jnp.concatenate(nums, axis=0)
    den = den + jnp.concatenate(dens, axis=0)

    out_ref[0] = num / den


def _fused_attention(q_s, k_s, v_s, kpi, vpi, sig, blk):
    grid = (BH, N // TQ)
    return pl.pallas_call(
        _attn_body,
        grid=grid,
        in_specs=[
            pl.BlockSpec((1, TQ, E), lambda bh, t: (bh, t, 0)),
            pl.BlockSpec((1, TQ, E), lambda bh, t: (bh, t, 0)),
            pl.BlockSpec((1, TQ, E), lambda bh, t: (bh, t, 0)),
            pl.BlockSpec((1, SAMPLE_SIZE, E), lambda bh, t: (bh, 0, 0)),
            pl.BlockSpec((1, SAMPLE_SIZE, E), lambda bh, t: (bh, 0, 0)),
            pl.BlockSpec((1, 1, SAMPLE_SIZE), lambda bh, t: (bh, 0, 0)),
            pl.BlockSpec((1, 1, SAMPLE_SIZE), lambda bh, t: (bh, 0, 0)),
        ],
        out_specs=pl.BlockSpec((1, TQ, E), lambda bh, t: (bh, t, 0)),
        out_shape=jax.ShapeDtypeStruct((BH, N, E), jnp.float32),
    )(q_s, k_s, v_s, kpi, vpi, sig, blk)


def _power_method(A):
    x = jax.random.normal(jax.random.key(42), (A.shape[0], A.shape[1], A.shape[3]),
                          dtype=A.dtype)
    x = x / jnp.linalg.norm(x, axis=2, keepdims=True)
    y = x
    for _ in range(32):
        y = jnp.einsum('bhnm,bhm->bhn', A, x)
        x = y / jnp.linalg.norm(y, axis=2, keepdims=True)
    return jnp.linalg.norm(y, axis=2)


def _take_rows(mat, idx):
    return jnp.take_along_axis(mat, idx[..., None], axis=2)


def kernel(query, key, value, proj_dir):
    softmax_temp = 1.0 / math.sqrt(E)
    scale = math.sqrt(softmax_temp)
    q = query * scale
    k = key * scale

    enc = (2 ** jnp.arange(NUM_PROJS)).reshape(1, 1, 1, -1)

    def _hash(mat):
        bits = jnp.matmul(mat, proj_dir) > 0
        g = (bits * enc).sum(-1)
        return g ^ (g >> 1)

    K_sort_idx = jnp.argsort(_hash(k), axis=2)
    Q_sort_idx = jnp.argsort(_hash(q), axis=2)

    value_aug = jnp.concatenate(
        [value, jnp.ones(value.shape[:3] + (1,), dtype=value.dtype)], axis=3)
    value_sorted = _take_rows(value_aug, K_sort_idx)
    Gram_V = jnp.einsum('bhnt,bhnd->bhtd', value_sorted, value_sorted)
    V_norm = _power_method(Gram_V)[:, :, None]
    P = jnp.linalg.norm(value_sorted, axis=3) / V_norm
    P = P + 1.0 / N
    P = P / jnp.sum(jnp.abs(P), axis=2, keepdims=True)
    logits = jnp.log(P.reshape(-1, N))
    idx = jax.random.categorical(
        jax.random.key(7),
        jnp.broadcast_to(logits[:, None, :], (BH, SAMPLE_SIZE, N)), axis=-1)
    idx = idx.reshape(B, H, SAMPLE_SIZE)

    orig_idx = jnp.take_along_axis(K_sort_idx, idx, axis=2)
    kpi = _take_rows(k, orig_idx)
    vpi = _take_rows(value, orig_idx)
    ppi = jnp.take_along_axis(P, idx, axis=2)
    sig = 1.0 / (ppi * SAMPLE_SIZE)
    blk = (idx // BUCKET_SIZE).astype(jnp.int32)

    q_s = _take_rows(q, Q_sort_idx)
    k_s = _take_rows(k, K_sort_idx)
    v_s = _take_rows(value, K_sort_idx)

    out_s = _fused_attention(
        q_s.reshape(BH, N, E), k_s.reshape(BH, N, E), v_s.reshape(BH, N, E),
        kpi.reshape(BH, SAMPLE_SIZE, E), vpi.reshape(BH, SAMPLE_SIZE, E),
        sig.reshape(BH, 1, SAMPLE_SIZE).astype(jnp.float32),
        blk.reshape(BH, 1, SAMPLE_SIZE))

    inv = jnp.zeros((BH * N,), jnp.int32).at[
        (Q_sort_idx.reshape(BH, N)
         + N * jnp.arange(BH, dtype=jnp.int32)[:, None]).reshape(-1)
    ].set(jnp.tile(jnp.arange(N, dtype=jnp.int32), BH)).reshape(B, H, N)
    out = _take_rows(out_s.reshape(B, H, N, E), inv)
    return out
